# Initial kernel scaffold; baseline (speedup 1.0000x reference)
import jax
import jax.numpy as jnp
from jax.experimental import pallas as pl


def kernel(inputs, w0, w, V):
    raise NotImplementedError("write your pallas kernel here")


# speedup vs baseline: 1.3489x; 1.3489x over previous
import functools

import jax
import jax.numpy as jnp
from jax import lax
from jax.experimental import pallas as pl
from jax.experimental.pallas import tpu as pltpu
from jax.experimental.pallas import tpu_sc as plsc

B = 16384
F = 26
K = 16
NC = 2
NS = 16
NW = NC * NS
BW = B // NW
CHUNK = 128
CHW = BW * F // CHUNK
G = 64
NG = BW // G
NCHG = G * F // CHUNK


def _fm_body(idx_hbm, w_hbm, v_hbm, out_hbm, idx_v, vbuf, wbuf, obuf, sem):
    wid = lax.axis_index("s") * NC + lax.axis_index("c")
    pltpu.sync_copy(idx_hbm.at[wid], idx_v)

    def group(g, carry):
        cbase = g * NCHG
        copies = []
        for j in range(NCHG):
            ixrow = idx_v.at[cbase + j]
            copies.append(pltpu.async_copy(
                v_hbm.at[ixrow], vbuf.at[pl.ds(j * CHUNK, CHUNK)], sem))
            copies.append(pltpu.async_copy(
                w_hbm.at[ixrow], wbuf.at[pl.ds(j * CHUNK, CHUNK)], sem))
        for c in copies:
            c.wait()

        lanes = lax.iota(jnp.int32, 16)

        def tile(t, c2):
            def sample(s16, rvec):
                s = t * 16 + s16
                base = s * F
                v0 = vbuf[base, :]
                sv = v0
                ssv = v0 * v0
                for f in range(1, F):
                    v = vbuf[base + f, :]
                    sv = sv + v
                    ssv = ssv + v * v
                wa = wbuf[pl.ds(base, 16)]
                wb = wbuf[pl.ds(base + 16, 16)]
                wbm = jnp.where(lanes < F - 16, wb, 0.0)
                total = 0.5 * (sv * sv - ssv) + wa + wbm
                for sh in (8, 4, 2, 1):
                    perm = (lanes + sh) & 15
                    total = total + total.at[perm].get(
                        mode="promise_in_bounds")
                return jnp.where(lanes == s16, total, rvec)

            rvec = lax.fori_loop(0, 16, sample,
                                 jnp.zeros((16,), jnp.float32))
            obuf[pl.ds(g * G + t * 16, 16)] = rvec
            return c2

        return lax.fori_loop(0, G // 16, tile, carry)

    lax.fori_loop(0, NG, group, 0)
    pltpu.sync_copy(obuf, out_hbm.at[pl.ds(wid * BW, BW)])


@functools.partial(jax.jit, static_argnames=())
def _fm_call(idx, w_flat, V):
    mesh = plsc.VectorSubcoreMesh(core_axis_name="c", subcore_axis_name="s",
                                  num_cores=NC, num_subcores=NS)
    run = pl.kernel(
        _fm_body,
        out_type=jax.ShapeDtypeStruct((B,), jnp.float32),
        mesh=mesh,
        scratch_types=[
            pltpu.VMEM((CHW, CHUNK), jnp.int32),
            pltpu.VMEM((G * F, K), jnp.float32),
            pltpu.VMEM((G * F + 16,), jnp.float32),
            pltpu.VMEM((BW,), jnp.float32),
            pltpu.SemaphoreType.DMA,
        ],
        compiler_params=pltpu.CompilerParams(use_tc_tiling_on_sc=False),
    )
    return run(idx, w_flat, V)


def kernel(inputs, w0, w, V):
    idx = inputs.astype(jnp.int32).reshape(NW, CHW, CHUNK)
    w_flat = w.reshape(w.shape[0])
    out = _fm_call(idx, w_flat, V)
    return out[:, None] + w0


# speedup vs baseline: 1.3805x; 1.0234x over previous
import functools

import jax
import jax.numpy as jnp
from jax import lax
from jax.experimental import pallas as pl
from jax.experimental.pallas import tpu as pltpu
from jax.experimental.pallas import tpu_sc as plsc

B = 16384
F = 26
K = 16
NC = 2
NS = 16
NW = NC * NS
BW = B // NW
CHUNK = 128
CHW = BW * F // CHUNK
G = 64
NG = BW // G
NCHG = G * F // CHUNK


def _fm_body(idx_hbm, w_hbm, v_hbm, out_hbm, idx_v,
             vbuf0, wbuf0, vbuf1, wbuf1, obuf, sem0, sem1):
    wid = lax.axis_index("s") * NC + lax.axis_index("c")
    pltpu.sync_copy(idx_hbm.at[wid], idx_v)

    def dmas(g, vbuf, wbuf, sem):
        cbase = g * NCHG
        out = []
        for j in range(NCHG):
            ixrow = idx_v.at[cbase + j]
            out.append(pltpu.make_async_copy(
                v_hbm.at[ixrow], vbuf.at[pl.ds(j * CHUNK, CHUNK)], sem))
            out.append(pltpu.make_async_copy(
                w_hbm.at[ixrow], wbuf.at[pl.ds(j * CHUNK, CHUNK)], sem))
        return out

    def fire(g, vbuf, wbuf, sem):
        for c in dmas(g, vbuf, wbuf, sem):
            c.start()

    def drain(g, vbuf, wbuf, sem):
        for c in dmas(g, vbuf, wbuf, sem):
            c.wait()

    def compute(g, vbuf, wbuf):
        lanes = lax.iota(jnp.int32, 16)

        def tile(t, c2):
            def sample(s16, rvec):
                base = (t * 16 + s16) * F
                v0 = vbuf[base, :]
                sv = v0
                ssv = v0 * v0
                for f in range(1, F):
                    v = vbuf[base + f, :]
                    sv = sv + v
                    ssv = ssv + v * v
                wa = wbuf[pl.ds(base, 16)]
                wb = wbuf[pl.ds(base + 16, 16)]
                wbm = jnp.where(lanes < F - 16, wb, 0.0)
                total = 0.5 * (sv * sv - ssv) + wa + wbm
                for sh in (8, 4, 2, 1):
                    perm = (lanes + sh) & 15
                    total = total + total.at[perm].get(
                        mode="promise_in_bounds")
                return jnp.where(lanes == s16, total, rvec)

            rvec = lax.fori_loop(0, 16, sample,
                                 jnp.zeros((16,), jnp.float32))
            obuf[pl.ds(g * G + t * 16, 16)] = rvec
            return c2

        lax.fori_loop(0, G // 16, tile, 0)

    fire(0, vbuf0, wbuf0, sem0)

    def pair(p, carry):
        g0 = 2 * p
        fire(g0 + 1, vbuf1, wbuf1, sem1)
        drain(g0, vbuf0, wbuf0, sem0)
        compute(g0, vbuf0, wbuf0)

        @pl.when(g0 + 2 < NG)
        def _():
            fire(g0 + 2, vbuf0, wbuf0, sem0)

        drain(g0 + 1, vbuf1, wbuf1, sem1)
        compute(g0 + 1, vbuf1, wbuf1)
        return carry

    lax.fori_loop(0, NG // 2, pair, 0)
    pltpu.sync_copy(obuf, out_hbm.at[pl.ds(wid * BW, BW)])


@functools.partial(jax.jit, static_argnames=())
def _fm_call(idx, w_flat, V):
    mesh = plsc.VectorSubcoreMesh(core_axis_name="c", subcore_axis_name="s",
                                  num_cores=NC, num_subcores=NS)
    run = pl.kernel(
        _fm_body,
        out_type=jax.ShapeDtypeStruct((B,), jnp.float32),
        mesh=mesh,
        scratch_types=[
            pltpu.VMEM((CHW, CHUNK), jnp.int32),
            pltpu.VMEM((G * F, K), jnp.float32),
            pltpu.VMEM((G * F + 16,), jnp.float32),
            pltpu.VMEM((G * F, K), jnp.float32),
            pltpu.VMEM((G * F + 16,), jnp.float32),
            pltpu.VMEM((BW,), jnp.float32),
            pltpu.SemaphoreType.DMA,
            pltpu.SemaphoreType.DMA,
        ],
        compiler_params=pltpu.CompilerParams(use_tc_tiling_on_sc=False),
    )
    return run(idx, w_flat, V)


def kernel(inputs, w0, w, V):
    idx = inputs.astype(jnp.int32).reshape(NW, CHW, CHUNK)
    w_flat = w.reshape(w.shape[0])
    out = _fm_call(idx, w_flat, V)
    return out[:, None] + w0
